# baseline (speedup 1.0000x reference)
import functools

import jax
import jax.numpy as jnp
from jax.experimental import pallas as pl


def _attn_pool_kernel(x_ref, w1_ref, b1_ref, w2_ref, b2_ref, out_ref, wts_ref):
    x = x_ref[...]
    w1 = w1_ref[...]
    u = jax.lax.dot_general(
        x, w1, (((2,), (0,)), ((), ())), preferred_element_type=jnp.float32
    )
    h = jnp.tanh(u + b1_ref[...])
    w2 = w2_ref[...]
    b2 = b2_ref[0, 0]
    bb = x.shape[0]
    rows = [
        jax.lax.dot_general(
            w2, h[b], (((1,), (1,)), ((), ())),
            preferred_element_type=jnp.float32,
        )
        for b in range(bb)
    ]
    scores = jnp.concatenate(rows, axis=0) + b2
    m = jnp.max(scores, axis=1, keepdims=True)
    e = jnp.exp(scores - m)
    ssum = jnp.sum(e, axis=1, keepdims=True)
    w = e / ssum
    wts_ref[...] = w
    out_ref[...] = jax.lax.dot_general(
        w, x, (((1,), (1,)), ((0,), (0,))),
        preferred_element_type=jnp.float32,
    )


@functools.partial(jax.jit, static_argnames=())
def kernel(input_tensor, W1, b1, W2, b2):
    B, N, D = input_tensor.shape
    H = W1.shape[1]
    BB = 16

    b1r = b1.reshape(1, H)
    w2r = W2.reshape(1, H)
    b2r = b2.reshape(1, 1)

    out, wts = pl.pallas_call(
        _attn_pool_kernel,
        grid=(B // BB,),
        in_specs=[
            pl.BlockSpec((BB, N, D), lambda i: (i, 0, 0)),
            pl.BlockSpec((D, H), lambda i: (0, 0)),
            pl.BlockSpec((1, H), lambda i: (0, 0)),
            pl.BlockSpec((1, H), lambda i: (0, 0)),
            pl.BlockSpec((1, 1), lambda i: (0, 0)),
        ],
        out_specs=[
            pl.BlockSpec((BB, D), lambda i: (i, 0)),
            pl.BlockSpec((BB, N), lambda i: (i, 0)),
        ],
        out_shape=[
            jax.ShapeDtypeStruct((B, D), jnp.float32),
            jax.ShapeDtypeStruct((B, N), jnp.float32),
        ],
    )(input_tensor, W1, b1r, w2r, b2r)
    return out, wts


# speedup vs baseline: 1.0921x; 1.0921x over previous
import functools

import jax
import jax.numpy as jnp
from jax.experimental import pallas as pl


def _attn_pool_kernel(x_ref, w1_ref, b1_ref, w2_ref, b2_ref, out_ref, wts_ref):
    x = x_ref[...]
    w1 = w1_ref[...]
    u = jax.lax.dot_general(
        x, w1, (((2,), (0,)), ((), ())), preferred_element_type=jnp.float32
    )
    h = jnp.tanh(u + b1_ref[...])
    w2 = w2_ref[...]
    b2 = b2_ref[0, 0]
    bb = x.shape[0]
    rows = [
        jax.lax.dot_general(
            w2, h[b], (((1,), (1,)), ((), ())),
            preferred_element_type=jnp.float32,
        )
        for b in range(bb)
    ]
    scores = jnp.concatenate(rows, axis=0) + b2
    m = jnp.max(scores, axis=1, keepdims=True)
    e = jnp.exp(scores - m)
    ssum = jnp.sum(e, axis=1, keepdims=True)
    w = e / ssum
    wts_ref[...] = w
    out_ref[...] = jax.lax.dot_general(
        w, x, (((1,), (1,)), ((0,), (0,))),
        preferred_element_type=jnp.float32,
    )


@functools.partial(jax.jit, static_argnames=())
def kernel(input_tensor, W1, b1, W2, b2):
    B, N, D = input_tensor.shape
    H = W1.shape[1]
    BB = 32

    b1r = b1.reshape(1, H)
    w2r = W2.reshape(1, H)
    b2r = b2.reshape(1, 1)

    out, wts = pl.pallas_call(
        _attn_pool_kernel,
        grid=(B // BB,),
        in_specs=[
            pl.BlockSpec((BB, N, D), lambda i: (i, 0, 0)),
            pl.BlockSpec((D, H), lambda i: (0, 0)),
            pl.BlockSpec((1, H), lambda i: (0, 0)),
            pl.BlockSpec((1, H), lambda i: (0, 0)),
            pl.BlockSpec((1, 1), lambda i: (0, 0)),
        ],
        out_specs=[
            pl.BlockSpec((BB, D), lambda i: (i, 0)),
            pl.BlockSpec((BB, N), lambda i: (i, 0)),
        ],
        out_shape=[
            jax.ShapeDtypeStruct((B, D), jnp.float32),
            jax.ShapeDtypeStruct((B, N), jnp.float32),
        ],
    )(input_tensor, W1, b1r, w2r, b2r)
    return out, wts
